# Initial kernel scaffold; baseline (speedup 1.0000x reference)
import functools

import jax
import jax.numpy as jnp
from jax import lax
from jax.experimental import pallas as pl
from jax.experimental.pallas import tpu as pltpu
from jax.experimental.pallas import tpu_sc as plsc

N = 10000
E = 320000
F = 128
HEADS = 8
HID = 16
NCLS = 40
FH = 64
HH = 4
DW = 16
NB = 10
BR = N // NB
K = 128
NBLK = E // K
NS = 16
BASE_BLK = NBLK // NS
EXTRA = NBLK - BASE_BLK * NS
RPT = N // NS

_f32 = jnp.float32


def _tc_embed_body(x_ref, w_ref, am_ref,
                   xw0_ref, xw1_ref, as0_ref, as1_ref, ad0_ref, ad1_ref):
    xw = jnp.dot(x_ref[...], w_ref[...], preferred_element_type=_f32)
    a = jnp.dot(xw, am_ref[...], preferred_element_type=_f32)
    xw0_ref[...] = xw[:, :FH]
    xw1_ref[...] = xw[:, FH:]
    as0_ref[...] = a[:, 0:HH]
    as1_ref[...] = a[:, HH:2 * HH]
    ad0_ref[...] = a[:, 2 * HH:3 * HH]
    ad1_ref[...] = a[:, 3 * HH:4 * HH]


def _normalize(o_ref, d_ref, er_ref):
    den = jnp.dot(d_ref[...], er_ref[...], preferred_element_type=_f32)
    return o_ref[...] / (den + 1e-16)


def _tc_mid_body(o0_ref, o1_ref, d0_ref, d1_ref, er_ref, b_ref, w_ref, am_ref,
                 xw0_ref, xw1_ref, as0_ref, as1_ref, ad0_ref, ad1_ref):
    h0 = _normalize(o0_ref, d0_ref, er_ref) + b_ref[0:1, :FH]
    h1 = _normalize(o1_ref, d1_ref, er_ref) + b_ref[0:1, FH:]
    h0 = jnp.where(h0 > 0, h0, jnp.expm1(h0))
    h1 = jnp.where(h1 > 0, h1, jnp.expm1(h1))
    xw = (jnp.dot(h0, w_ref[:FH, :], preferred_element_type=_f32)
          + jnp.dot(h1, w_ref[FH:, :], preferred_element_type=_f32))
    a = jnp.dot(xw, am_ref[...], preferred_element_type=_f32)
    xw0_ref[...] = xw[:, :FH]
    xw1_ref[...] = xw[:, FH:]
    as0_ref[...] = a[:, 0:HH]
    as1_ref[...] = a[:, HH:2 * HH]
    ad0_ref[...] = a[:, 2 * HH:3 * HH]
    ad1_ref[...] = a[:, 3 * HH:4 * HH]


def _tc_out_body(o0_ref, o1_ref, d0_ref, d1_ref, er_ref, b_ref, wc_ref, bc_ref,
                 out_ref):
    h0 = _normalize(o0_ref, d0_ref, er_ref) + b_ref[0:1, :FH]
    h1 = _normalize(o1_ref, d1_ref, er_ref) + b_ref[0:1, FH:]
    out_ref[...] = (jnp.dot(h0, wc_ref[:FH, :], preferred_element_type=_f32)
                    + jnp.dot(h1, wc_ref[FH:, :], preferred_element_type=_f32)
                    + bc_ref[0:1, :])


def _row_spec(width):
    return pl.BlockSpec((BR, width), lambda i: (i, 0))


def _full_spec(shape):
    return pl.BlockSpec(shape, lambda i: (0, 0))


_EMBED_CALL = None
_MID_CALL = None
_OUT_CALL = None


def _tc_calls():
    global _EMBED_CALL, _MID_CALL, _OUT_CALL
    if _EMBED_CALL is not None:
        return _EMBED_CALL, _MID_CALL, _OUT_CALL
    node_outs = ([jax.ShapeDtypeStruct((N, FH), _f32)] * 2
                 + [jax.ShapeDtypeStruct((N, HH), _f32)] * 4)
    node_out_specs = [_row_spec(FH)] * 2 + [_row_spec(HH)] * 4
    _EMBED_CALL = pl.pallas_call(
        _tc_embed_body,
        grid=(NB,),
        in_specs=[_row_spec(F), _full_spec((F, F)), _full_spec((F, 2 * HEADS))],
        out_specs=node_out_specs,
        out_shape=node_outs,
    )
    _MID_CALL = pl.pallas_call(
        _tc_mid_body,
        grid=(NB,),
        in_specs=[_row_spec(FH), _row_spec(FH), _row_spec(DW), _row_spec(DW),
                  _full_spec((DW, FH)), _full_spec((1, F)),
                  _full_spec((F, F)), _full_spec((F, 2 * HEADS))],
        out_specs=node_out_specs,
        out_shape=node_outs,
    )
    _OUT_CALL = pl.pallas_call(
        _tc_out_body,
        grid=(NB,),
        in_specs=[_row_spec(FH), _row_spec(FH), _row_spec(DW), _row_spec(DW),
                  _full_spec((DW, FH)), _full_spec((1, F)),
                  _full_spec((F, NCLS)), _full_spec((1, NCLS))],
        out_specs=[_row_spec(NCLS)],
        out_shape=[jax.ShapeDtypeStruct((N, NCLS), _f32)],
    )
    return _EMBED_CALL, _MID_CALL, _OUT_CALL


def _sc_edge_body(xw0, xw1, as0, as1, ad0, ad1, srcv, dstv, z64, z16,
                  out0, out1, den0, den1,
                  tb, ob, db, at_s, at_d, sidx, didx, rows, dstage, evals,
                  gsem):
    c = lax.axis_index("c")
    s = lax.axis_index("s")
    r0 = s * RPT

    @pl.when(c == 0)
    def _():
        pltpu.sync_copy(xw0.at[pl.ds(r0, RPT)], tb.at[pl.ds(r0, RPT)])
        pltpu.sync_copy(as0, at_s)
        pltpu.sync_copy(ad0, at_d)

    @pl.when(c == 1)
    def _():
        pltpu.sync_copy(xw1.at[pl.ds(r0, RPT)], tb.at[pl.ds(r0, RPT)])
        pltpu.sync_copy(as1, at_s)
        pltpu.sync_copy(ad1, at_d)

    pltpu.sync_copy(z64.at[pl.ds(r0, RPT)], ob.at[pl.ds(r0, RPT)])
    pltpu.sync_copy(z16.at[pl.ds(r0, RPT)], db.at[pl.ds(r0, RPT)])

    zv = jnp.zeros((16,), _f32)
    for k in range(K):
        dstage[k, :] = zv

    plsc.subcore_barrier()

    nblk = BASE_BLK + jnp.where(s < EXTRA, 1, 0)
    blk0 = s * BASE_BLK + jnp.minimum(s, EXTRA)

    def block_body(b, carry):
        off = (blk0 + b) * K
        pltpu.sync_copy(srcv.at[pl.ds(off, K)], sidx)
        pltpu.sync_copy(dstv.at[pl.ds(off, K)], didx)
        gather = pltpu.async_copy(tb.at[sidx], rows, gsem)
        for j in range(K // 16):
            sv = sidx[pl.ds(j * 16, 16)]
            dv = didx[pl.ds(j * 16, 16)]
            kidx = lax.iota(jnp.int32, 16) + (j * 16)
            for h in range(HH):
                hv = jnp.full((16,), h, jnp.int32)
                av = (plsc.load_gather(at_s, [sv, hv])
                      + plsc.load_gather(at_d, [dv, hv]))
                av = jnp.where(av > 0, av, av * 0.2)
                ev = jnp.exp(av)
                evals[h, pl.ds(j * 16, 16)] = ev
                plsc.store_scatter(dstage, [kidx, hv], ev)
        gather.wait()

        def scale_body(k, acc):
            for h in range(HH):
                e = evals[h, k]
                rows[k, pl.ds(h * HID, HID)] = rows[k, pl.ds(h * HID, HID)] * e
            return acc

        lax.fori_loop(0, K, scale_body, 0)

        pltpu.sync_copy(rows, ob.at[didx], add=True)
        pltpu.sync_copy(dstage, db.at[didx], add=True)
        return carry

    lax.fori_loop(0, nblk, block_body, 0)

    plsc.subcore_barrier()

    @pl.when(c == 0)
    def _():
        pltpu.sync_copy(ob.at[pl.ds(r0, RPT)], out0.at[pl.ds(r0, RPT)])
        pltpu.sync_copy(db.at[pl.ds(r0, RPT)], den0.at[pl.ds(r0, RPT)])

    @pl.when(c == 1)
    def _():
        pltpu.sync_copy(ob.at[pl.ds(r0, RPT)], out1.at[pl.ds(r0, RPT)])
        pltpu.sync_copy(db.at[pl.ds(r0, RPT)], den1.at[pl.ds(r0, RPT)])


_SC_CALL = None


def _sc_call():
    global _SC_CALL
    if _SC_CALL is None:
        mesh = plsc.VectorSubcoreMesh(core_axis_name="c", subcore_axis_name="s")
        _SC_CALL = pl.kernel(
            _sc_edge_body,
            out_type=[jax.ShapeDtypeStruct((N, FH), _f32),
                      jax.ShapeDtypeStruct((N, FH), _f32),
                      jax.ShapeDtypeStruct((N, DW), _f32),
                      jax.ShapeDtypeStruct((N, DW), _f32)],
            mesh=mesh,
            scratch_types=[
                pltpu.VMEM_SHARED((N, FH), _f32),
                pltpu.VMEM_SHARED((N, FH), _f32),
                pltpu.VMEM_SHARED((N, DW), _f32),
                pltpu.VMEM((N, HH), _f32),
                pltpu.VMEM((N, HH), _f32),
                pltpu.VMEM((K,), jnp.int32),
                pltpu.VMEM((K,), jnp.int32),
                pltpu.VMEM((K, FH), _f32),
                pltpu.VMEM((K, DW), _f32),
                pltpu.VMEM((HH, K), _f32),
                pltpu.SemaphoreType.DMA,
            ],
        )
    return _SC_CALL


def _att_mats(att_src, att_dst):
    eye = jnp.eye(HEADS, dtype=_f32)
    a_s = (att_src[0][:, :, None] * eye[:, None, :]).reshape(F, HEADS)
    a_d = (att_dst[0][:, :, None] * eye[:, None, :]).reshape(F, HEADS)
    return jnp.concatenate([a_s, a_d], axis=1)


@jax.jit
def kernel(x, edge_index, W0, att_src0, att_dst0, b0,
           W1, att_src1, att_dst1, b1, Wc, bc):
    src = edge_index[0]
    dst = edge_index[1]
    am0 = _att_mats(att_src0, att_dst0)
    am1 = _att_mats(att_src1, att_dst1)
    er = jnp.concatenate(
        [jnp.kron(jnp.eye(HH, dtype=_f32), jnp.ones((1, HID), _f32)),
         jnp.zeros((DW - HH, FH), _f32)], axis=0)
    z64 = jnp.zeros((N, FH), _f32)
    z16 = jnp.zeros((N, DW), _f32)

    embed, mid, outk = _tc_calls()
    sc_edge = _sc_call()

    xw0a, xw0b, as0a, as0b, ad0a, ad0b = embed(x, W0, am0)
    o0a, o0b, d0a, d0b = sc_edge(xw0a, xw0b, as0a, as0b, ad0a, ad0b,
                                 src, dst, z64, z16)
    xw1a, xw1b, as1a, as1b, ad1a, ad1b = mid(
        o0a, o0b, d0a, d0b, er, b0[None, :], W1, am1)
    o1a, o1b, d1a, d1b = sc_edge(xw1a, xw1b, as1a, as1b, ad1a, ad1b,
                                 src, dst, z64, z16)
    (logits,) = outk(o1a, o1b, d1a, d1b, er, b1[None, :], Wc, bc[None, :])
    return logits


# speedup vs baseline: 51.9719x; 51.9719x over previous
import jax
import jax.numpy as jnp
from jax import lax
from jax.experimental import pallas as pl
from jax.experimental.pallas import tpu as pltpu
from jax.experimental.pallas import tpu_sc as plsc

N = 10000
E = 320000
F = 128
HEADS = 8
HID = 16
NCLS = 40
FH = 64
HH = 4
AW = 80
ONES_OFF = 64
DW = 16
NB = 10
BR = N // NB
K = 128
NBLK = E // K
NS = 16
BASE_BLK = NBLK // NS
EXTRA = NBLK - BASE_BLK * NS
RPT = 624
NTAIL = N - NS * RPT

_f32 = jnp.float32


def _vtake(v, idx):
    dn = lax.GatherDimensionNumbers(offset_dims=(), collapsed_slice_dims=(0,),
                                    start_index_map=(0,))
    return lax.gather(v, idx[:, None], dn, slice_sizes=(1,),
                      mode=lax.GatherScatterMode.PROMISE_IN_BOUNDS)


def _augment(xw_half):
    ones = jnp.ones((xw_half.shape[0], HH), _f32)
    pad = jnp.zeros((xw_half.shape[0], AW - ONES_OFF - HH), _f32)
    return jnp.concatenate([xw_half, ones, pad], axis=1)


def _split_outs(xw, a, xw0_ref, xw1_ref, at0_ref, at1_ref):
    xw0_ref[...] = _augment(xw[:, :FH])
    xw1_ref[...] = _augment(xw[:, FH:])
    zpad = jnp.zeros((a.shape[0], 8), _f32)
    at0_ref[...] = jnp.concatenate(
        [a[:, 0:HH], a[:, 2 * HH:3 * HH], zpad], axis=1)
    at1_ref[...] = jnp.concatenate(
        [a[:, HH:2 * HH], a[:, 3 * HH:4 * HH], zpad], axis=1)


def _tc_embed_body(x_ref, w_ref, am_ref, xw0_ref, xw1_ref, at0_ref, at1_ref):
    xw = jnp.dot(x_ref[...], w_ref[...], preferred_element_type=_f32)
    a = jnp.dot(xw, am_ref[...], preferred_element_type=_f32)
    _split_outs(xw, a, xw0_ref, xw1_ref, at0_ref, at1_ref)


def _normalize(o_ref, er_ref):
    den = jnp.dot(o_ref[:, ONES_OFF:], er_ref[...], preferred_element_type=_f32)
    return o_ref[:, :FH] / (den + 1e-16)


def _tc_mid_body(o0_ref, o1_ref, er_ref, b_ref, w_ref, am_ref,
                 xw0_ref, xw1_ref, at0_ref, at1_ref):
    h0 = _normalize(o0_ref, er_ref) + b_ref[0:1, :FH]
    h1 = _normalize(o1_ref, er_ref) + b_ref[0:1, FH:]
    h0 = jnp.where(h0 > 0, h0, jnp.exp(h0) - 1.0)
    h1 = jnp.where(h1 > 0, h1, jnp.exp(h1) - 1.0)
    xw = (jnp.dot(h0, w_ref[:FH, :], preferred_element_type=_f32)
          + jnp.dot(h1, w_ref[FH:, :], preferred_element_type=_f32))
    a = jnp.dot(xw, am_ref[...], preferred_element_type=_f32)
    _split_outs(xw, a, xw0_ref, xw1_ref, at0_ref, at1_ref)


def _tc_out_body(o0_ref, o1_ref, er_ref, b_ref, wc_ref, bc_ref, out_ref):
    h0 = _normalize(o0_ref, er_ref) + b_ref[0:1, :FH]
    h1 = _normalize(o1_ref, er_ref) + b_ref[0:1, FH:]
    out_ref[...] = (jnp.dot(h0, wc_ref[:FH, :], preferred_element_type=_f32)
                    + jnp.dot(h1, wc_ref[FH:, :], preferred_element_type=_f32)
                    + bc_ref[0:1, :])


def _row_spec(width):
    return pl.BlockSpec((BR, width), lambda i: (i, 0))


def _full_spec(shape):
    return pl.BlockSpec(shape, lambda i: (0, 0))


_CALLS = None


def _tc_calls():
    global _CALLS
    if _CALLS is not None:
        return _CALLS
    node_outs = ([jax.ShapeDtypeStruct((N, AW), _f32)] * 2
                 + [jax.ShapeDtypeStruct((N, DW), _f32)] * 2)
    node_out_specs = [_row_spec(AW)] * 2 + [_row_spec(DW)] * 2
    embed = pl.pallas_call(
        _tc_embed_body,
        grid=(NB,),
        in_specs=[_row_spec(F), _full_spec((F, F)), _full_spec((F, 2 * HEADS))],
        out_specs=node_out_specs,
        out_shape=node_outs,
    )
    mid = pl.pallas_call(
        _tc_mid_body,
        grid=(NB,),
        in_specs=[_row_spec(AW), _row_spec(AW),
                  _full_spec((DW, FH)), _full_spec((1, F)),
                  _full_spec((F, F)), _full_spec((F, 2 * HEADS))],
        out_specs=node_out_specs,
        out_shape=node_outs,
    )
    outk = pl.pallas_call(
        _tc_out_body,
        grid=(NB,),
        in_specs=[_row_spec(AW), _row_spec(AW),
                  _full_spec((DW, FH)), _full_spec((1, F)),
                  _full_spec((F, NCLS)), _full_spec((1, NCLS))],
        out_specs=[_row_spec(NCLS)],
        out_shape=[jax.ShapeDtypeStruct((N, NCLS), _f32)],
    )
    _CALLS = (embed, mid, outk)
    return _CALLS


def _sc_edge_body(xw0, xw1, at0, at1, srcv, dstv, z80,
                  out0, out1,
                  tb, ob, asp, sidx, didx, rows, asr, adr,
                  gsem, asem, dsem):
    c = lax.axis_index("c")
    s = lax.axis_index("s")
    r0 = s * RPT

    def part_copy(src, dst):
        pltpu.sync_copy(src.at[pl.ds(r0, RPT)], dst.at[pl.ds(r0, RPT)])

        @pl.when(s == NS - 1)
        def _():
            pltpu.sync_copy(src.at[pl.ds(NS * RPT, NTAIL)],
                            dst.at[pl.ds(NS * RPT, NTAIL)])

    @pl.when(c == 0)
    def _():
        part_copy(xw0, tb)
        part_copy(at0, asp)

    @pl.when(c == 1)
    def _():
        part_copy(xw1, tb)
        part_copy(at1, asp)

    part_copy(z80, ob)

    plsc.subcore_barrier()

    lanes = lax.iota(jnp.int32, 16)
    shift4 = (lanes + HH) & 15
    lmask = lanes < HH

    nblk = BASE_BLK + jnp.where(s < EXTRA, 1, 0)
    blk0 = s * BASE_BLK + jnp.minimum(s, EXTRA)

    def block_body(b, carry):
        off = (blk0 + b) * K
        pltpu.sync_copy(srcv.at[pl.ds(off, K)], sidx)
        pltpu.sync_copy(dstv.at[pl.ds(off, K)], didx)
        g1 = pltpu.async_copy(tb.at[sidx], rows, gsem)
        g2 = pltpu.async_copy(asp.at[sidx], asr, asem)
        g3 = pltpu.async_copy(asp.at[didx], adr, dsem)
        g2.wait()
        g3.wait()
        g1.wait()

        def scale_body(k, acc):
            al = asr[k, :] + _vtake(adr[k, :], shift4)
            al = jnp.where(al > 0, al, al * 0.2)
            ev = jnp.exp(al)
            rows[k, pl.ds(ONES_OFF, 16)] = jnp.where(lmask, ev, 0.0)
            for h in range(HH):
                bh = _vtake(ev, jnp.full((16,), h, jnp.int32))
                rows[k, pl.ds(h * HID, HID)] = (
                    rows[k, pl.ds(h * HID, HID)] * bh)
            return acc

        lax.fori_loop(0, K, scale_body, 0)

        pltpu.sync_copy(rows, ob.at[didx], add=True)
        return carry

    lax.fori_loop(0, nblk, block_body, 0)

    plsc.subcore_barrier()

    @pl.when(c == 0)
    def _():
        part_copy(ob, out0)

    @pl.when(c == 1)
    def _():
        part_copy(ob, out1)


_SC_CALL = None


def _sc_call():
    global _SC_CALL
    if _SC_CALL is None:
        mesh = plsc.VectorSubcoreMesh(core_axis_name="c", subcore_axis_name="s")
        _SC_CALL = pl.kernel(
            _sc_edge_body,
            out_type=[jax.ShapeDtypeStruct((N, AW), _f32),
                      jax.ShapeDtypeStruct((N, AW), _f32)],
            mesh=mesh,
            compiler_params=pltpu.CompilerParams(use_tc_tiling_on_sc=False),
            scratch_types=[
                pltpu.VMEM_SHARED((N, AW), _f32),
                pltpu.VMEM_SHARED((N, AW), _f32),
                pltpu.VMEM_SHARED((N, DW), _f32),
                pltpu.VMEM((K,), jnp.int32),
                pltpu.VMEM((K,), jnp.int32),
                pltpu.VMEM((K, AW), _f32),
                pltpu.VMEM((K, DW), _f32),
                pltpu.VMEM((K, DW), _f32),
                pltpu.SemaphoreType.DMA,
                pltpu.SemaphoreType.DMA,
                pltpu.SemaphoreType.DMA,
            ],
        )
    return _SC_CALL


def _att_mats(att_src, att_dst):
    eye = jnp.eye(HEADS, dtype=_f32)
    a_s = (att_src[0][:, :, None] * eye[:, None, :]).reshape(F, HEADS)
    a_d = (att_dst[0][:, :, None] * eye[:, None, :]).reshape(F, HEADS)
    return jnp.concatenate([a_s, a_d], axis=1)


@jax.jit
def kernel(x, edge_index, W0, att_src0, att_dst0, b0,
           W1, att_src1, att_dst1, b1, Wc, bc):
    src = edge_index[0]
    dst = edge_index[1]
    am0 = _att_mats(att_src0, att_dst0)
    am1 = _att_mats(att_src1, att_dst1)
    er = jnp.concatenate(
        [jnp.kron(jnp.eye(HH, dtype=_f32), jnp.ones((1, HID), _f32)),
         jnp.zeros((DW - HH, FH), _f32)], axis=0)
    z80 = jnp.zeros((N, AW), _f32)

    embed, mid, outk = _tc_calls()
    sc_edge = _sc_call()

    xw0a, xw0b, at0a, at0b = embed(x, W0, am0)
    o0a, o0b = sc_edge(xw0a, xw0b, at0a, at0b, src, dst, z80)
    xw1a, xw1b, at1a, at1b = mid(o0a, o0b, er, b0[None, :], W1, am1)
    o1a, o1b = sc_edge(xw1a, xw1b, at1a, at1b, src, dst, z80)
    (logits,) = outk(o1a, o1b, er, b1[None, :], Wc, bc[None, :])
    return logits


# speedup vs baseline: 52.2997x; 1.0063x over previous
import jax
import jax.numpy as jnp
from jax import lax
from jax.experimental import pallas as pl
from jax.experimental.pallas import tpu as pltpu
from jax.experimental.pallas import tpu_sc as plsc

N = 10000
E = 320000
F = 128
HEADS = 8
HID = 16
NCLS = 40
FH = 64
HH = 4
DW = 16
NB = 10
BR = N // NB
K = 128
NBLK = E // K
NS = 16
MAIN = NBLK // NS
EXTRA = NBLK - MAIN * NS
NSLOT = 2
RPT = 624
NTAIL = N - NS * RPT

_f32 = jnp.float32


def _vtake(v, idx):
    dn = lax.GatherDimensionNumbers(offset_dims=(), collapsed_slice_dims=(0,),
                                    start_index_map=(0,))
    return lax.gather(v, idx[:, None], dn, slice_sizes=(1,),
                      mode=lax.GatherScatterMode.PROMISE_IN_BOUNDS)


def _split_outs(xw, a, xw0_ref, xw1_ref, at0_ref, at1_ref):
    xw0_ref[...] = xw[:, :FH]
    xw1_ref[...] = xw[:, FH:]
    zpad = jnp.zeros((a.shape[0], 8), _f32)
    at0_ref[...] = jnp.concatenate(
        [a[:, 0:HH], a[:, 2 * HH:3 * HH], zpad], axis=1)
    at1_ref[...] = jnp.concatenate(
        [a[:, HH:2 * HH], a[:, 3 * HH:4 * HH], zpad], axis=1)


def _tc_embed_body(x_ref, w_ref, am_ref, xw0_ref, xw1_ref, at0_ref, at1_ref):
    xw = jnp.dot(x_ref[...], w_ref[...], preferred_element_type=_f32)
    a = jnp.dot(xw, am_ref[...], preferred_element_type=_f32)
    _split_outs(xw, a, xw0_ref, xw1_ref, at0_ref, at1_ref)


def _normalize(o_ref, d_ref, er_ref):
    den = jnp.dot(d_ref[...], er_ref[...], preferred_element_type=_f32)
    return o_ref[...] / (den + 1e-16)


def _tc_mid_body(o0_ref, o1_ref, d0_ref, d1_ref, er_ref, b_ref, w_ref, am_ref,
                 xw0_ref, xw1_ref, at0_ref, at1_ref):
    h0 = _normalize(o0_ref, d0_ref, er_ref) + b_ref[0:1, :FH]
    h1 = _normalize(o1_ref, d1_ref, er_ref) + b_ref[0:1, FH:]
    h0 = jnp.where(h0 > 0, h0, jnp.exp(h0) - 1.0)
    h1 = jnp.where(h1 > 0, h1, jnp.exp(h1) - 1.0)
    xw = (jnp.dot(h0, w_ref[:FH, :], preferred_element_type=_f32)
          + jnp.dot(h1, w_ref[FH:, :], preferred_element_type=_f32))
    a = jnp.dot(xw, am_ref[...], preferred_element_type=_f32)
    _split_outs(xw, a, xw0_ref, xw1_ref, at0_ref, at1_ref)


def _tc_out_body(o0_ref, o1_ref, d0_ref, d1_ref, er_ref, b_ref, wc_ref,
                 bc_ref, out_ref):
    h0 = _normalize(o0_ref, d0_ref, er_ref) + b_ref[0:1, :FH]
    h1 = _normalize(o1_ref, d1_ref, er_ref) + b_ref[0:1, FH:]
    out_ref[...] = (jnp.dot(h0, wc_ref[:FH, :], preferred_element_type=_f32)
                    + jnp.dot(h1, wc_ref[FH:, :], preferred_element_type=_f32)
                    + bc_ref[0:1, :])


def _row_spec(width):
    return pl.BlockSpec((BR, width), lambda i: (i, 0))


def _full_spec(shape):
    return pl.BlockSpec(shape, lambda i: (0, 0))


_CALLS = None


def _tc_calls():
    global _CALLS
    if _CALLS is not None:
        return _CALLS
    node_outs = ([jax.ShapeDtypeStruct((N, FH), _f32)] * 2
                 + [jax.ShapeDtypeStruct((N, DW), _f32)] * 2)
    node_out_specs = [_row_spec(FH)] * 2 + [_row_spec(DW)] * 2
    embed = pl.pallas_call(
        _tc_embed_body,
        grid=(NB,),
        in_specs=[_row_spec(F), _full_spec((F, F)), _full_spec((F, 2 * HEADS))],
        out_specs=node_out_specs,
        out_shape=node_outs,
    )
    mid = pl.pallas_call(
        _tc_mid_body,
        grid=(NB,),
        in_specs=[_row_spec(FH), _row_spec(FH), _row_spec(DW), _row_spec(DW),
                  _full_spec((DW, FH)), _full_spec((1, F)),
                  _full_spec((F, F)), _full_spec((F, 2 * HEADS))],
        out_specs=node_out_specs,
        out_shape=node_outs,
    )
    outk = pl.pallas_call(
        _tc_out_body,
        grid=(NB,),
        in_specs=[_row_spec(FH), _row_spec(FH), _row_spec(DW), _row_spec(DW),
                  _full_spec((DW, FH)), _full_spec((1, F)),
                  _full_spec((F, NCLS)), _full_spec((1, NCLS))],
        out_specs=[_row_spec(NCLS)],
        out_shape=[jax.ShapeDtypeStruct((N, NCLS), _f32)],
    )
    _CALLS = (embed, mid, outk)
    return _CALLS


def _sc_edge_body(xw0, xw1, at0, at1, srcv, dstv, z64, z16,
                  out0, out1, den0, den1,
                  tb, ob, db, asp,
                  sidx_a, didx_a, rows_a, dst_a, asr_a, adr_a,
                  sidx_b, didx_b, rows_b, dst_b, asr_b, adr_b,
                  gsem_a, gsem_b):
    c = lax.axis_index("c")
    s = lax.axis_index("s")
    r0 = s * RPT

    sidxs = (sidx_a, sidx_b)
    didxs = (didx_a, didx_b)
    rowss = (rows_a, rows_b)
    dsts = (dst_a, dst_b)
    asrs = (asr_a, asr_b)
    adrs = (adr_a, adr_b)
    gsems = (gsem_a, gsem_b)

    def part_copy(src, dst):
        pltpu.sync_copy(src.at[pl.ds(r0, RPT)], dst.at[pl.ds(r0, RPT)])

        @pl.when(s == NS - 1)
        def _():
            pltpu.sync_copy(src.at[pl.ds(NS * RPT, NTAIL)],
                            dst.at[pl.ds(NS * RPT, NTAIL)])

    @pl.when(c == 0)
    def _():
        part_copy(xw0, tb)
        part_copy(at0, asp)

    @pl.when(c == 1)
    def _():
        part_copy(xw1, tb)
        part_copy(at1, asp)

    part_copy(z64, ob)
    part_copy(z16, db)

    plsc.subcore_barrier()

    lanes = lax.iota(jnp.int32, 16)
    shift4 = (lanes + HH) & 15
    lmask = lanes < HH

    blk0 = s * MAIN

    def get_idx(j, blk):
        off = blk * K
        pltpu.sync_copy(srcv.at[pl.ds(off, K)], sidxs[j])
        pltpu.sync_copy(dstv.at[pl.ds(off, K)], didxs[j])

    def start_gathers(j):
        pltpu.async_copy(tb.at[sidxs[j]], rowss[j], gsems[j])
        pltpu.async_copy(asp.at[sidxs[j]], asrs[j], gsems[j])
        pltpu.async_copy(asp.at[didxs[j]], adrs[j], gsems[j])

    def wait_gathers(j):
        pltpu.make_async_copy(tb.at[sidxs[j]], rowss[j], gsems[j]).wait()
        pltpu.make_async_copy(asp.at[sidxs[j]], asrs[j], gsems[j]).wait()
        pltpu.make_async_copy(asp.at[didxs[j]], adrs[j], gsems[j]).wait()

    def compute(j):
        rows_s = rowss[j]
        dst_s = dsts[j]
        asr_s = asrs[j]
        adr_s = adrs[j]

        def scale_body(k, acc):
            al = asr_s[k, :] + _vtake(adr_s[k, :], shift4)
            al = jnp.where(al > 0, al, al * 0.2)
            ev = jnp.exp(al)
            dst_s[k, :] = jnp.where(lmask, ev, 0.0)
            for h in range(HH):
                bh = _vtake(ev, jnp.full((16,), h, jnp.int32))
                rows_s[k, pl.ds(h * HID, HID)] = (
                    rows_s[k, pl.ds(h * HID, HID)] * bh)
            return acc

        lax.fori_loop(0, K, scale_body, 0)

    def scatter(j):
        pltpu.sync_copy(rowss[j], ob.at[didxs[j]], add=True)
        pltpu.sync_copy(dsts[j], db.at[didxs[j]], add=True)

    def round_body(r, carry):
        for j in range(NSLOT):
            get_idx(j, blk0 + NSLOT * r + j)
            start_gathers(j)
        for j in range(NSLOT):
            wait_gathers(j)
            compute(j)
            scatter(j)
        return carry

    lax.fori_loop(0, MAIN // NSLOT, round_body, 0)

    @pl.when(s < EXTRA)
    def _():
        get_idx(0, NS * MAIN + s)
        pltpu.sync_copy(tb.at[sidx_a], rows_a)
        pltpu.sync_copy(asp.at[sidx_a], asr_a)
        pltpu.sync_copy(asp.at[didx_a], adr_a)
        compute(0)
        scatter(0)

    plsc.subcore_barrier()

    @pl.when(c == 0)
    def _():
        part_copy(ob, out0)
        part_copy(db, den0)

    @pl.when(c == 1)
    def _():
        part_copy(ob, out1)
        part_copy(db, den1)


_SC_CALL = None


def _sc_call():
    global _SC_CALL
    if _SC_CALL is None:
        mesh = plsc.VectorSubcoreMesh(core_axis_name="c", subcore_axis_name="s")
        slot_bufs = [
            pltpu.VMEM((K,), jnp.int32),
            pltpu.VMEM((K,), jnp.int32),
            pltpu.VMEM((K, FH), _f32),
            pltpu.VMEM((K, DW), _f32),
            pltpu.VMEM((K, DW), _f32),
            pltpu.VMEM((K, DW), _f32),
        ]
        _SC_CALL = pl.kernel(
            _sc_edge_body,
            out_type=[jax.ShapeDtypeStruct((N, FH), _f32),
                      jax.ShapeDtypeStruct((N, FH), _f32),
                      jax.ShapeDtypeStruct((N, DW), _f32),
                      jax.ShapeDtypeStruct((N, DW), _f32)],
            mesh=mesh,
            compiler_params=pltpu.CompilerParams(use_tc_tiling_on_sc=False),
            scratch_types=(
                [pltpu.VMEM_SHARED((N, FH), _f32),
                 pltpu.VMEM_SHARED((N, FH), _f32),
                 pltpu.VMEM_SHARED((N, DW), _f32),
                 pltpu.VMEM_SHARED((N, DW), _f32)]
                + slot_bufs + slot_bufs
                + [pltpu.SemaphoreType.DMA, pltpu.SemaphoreType.DMA]
            ),
        )
    return _SC_CALL


def _att_mats(att_src, att_dst):
    eye = jnp.eye(HEADS, dtype=_f32)
    a_s = (att_src[0][:, :, None] * eye[:, None, :]).reshape(F, HEADS)
    a_d = (att_dst[0][:, :, None] * eye[:, None, :]).reshape(F, HEADS)
    return jnp.concatenate([a_s, a_d], axis=1)


@jax.jit
def kernel(x, edge_index, W0, att_src0, att_dst0, b0,
           W1, att_src1, att_dst1, b1, Wc, bc):
    src = edge_index[0]
    dst = edge_index[1]
    am0 = _att_mats(att_src0, att_dst0)
    am1 = _att_mats(att_src1, att_dst1)
    er = jnp.concatenate(
        [jnp.kron(jnp.eye(HH, dtype=_f32), jnp.ones((1, HID), _f32)),
         jnp.zeros((DW - HH, FH), _f32)], axis=0)
    z64 = jnp.zeros((N, FH), _f32)
    z16 = jnp.zeros((N, DW), _f32)

    embed, mid, outk = _tc_calls()
    sc_edge = _sc_call()

    xw0a, xw0b, at0a, at0b = embed(x, W0, am0)
    o0a, o0b, d0a, d0b = sc_edge(xw0a, xw0b, at0a, at0b, src, dst, z64, z16)
    xw1a, xw1b, at1a, at1b = mid(o0a, o0b, d0a, d0b, er, b0[None, :],
                                 W1, am1)
    o1a, o1b, d1a, d1b = sc_edge(xw1a, xw1b, at1a, at1b, src, dst, z64, z16)
    (logits,) = outk(o1a, o1b, d1a, d1b, er, b1[None, :], Wc, bc[None, :])
    return logits


# speedup vs baseline: 60.8355x; 1.1632x over previous
import jax
import jax.numpy as jnp
from jax import lax
from jax.experimental import pallas as pl
from jax.experimental.pallas import tpu as pltpu
from jax.experimental.pallas import tpu_sc as plsc

N = 10000
E = 320000
F = 128
HEADS = 8
HID = 16
NCLS = 40
FH = 64
HH = 4
DW = 16
NB = 10
BR = N // NB
K = 128
NBLK = E // K
NS = 16
MAIN = NBLK // NS
EXTRA = NBLK - MAIN * NS
NSLOT = 2
SLAB = 6
RPT = 624
NTAIL = N - NS * RPT

_f32 = jnp.float32


def _vtake(v, idx):
    dn = lax.GatherDimensionNumbers(offset_dims=(), collapsed_slice_dims=(0,),
                                    start_index_map=(0,))
    return lax.gather(v, idx[:, None], dn, slice_sizes=(1,),
                      mode=lax.GatherScatterMode.PROMISE_IN_BOUNDS)


def _split_outs(xw, a, xw0_ref, xw1_ref, at0_ref, at1_ref):
    xw0_ref[...] = xw[:, :FH]
    xw1_ref[...] = xw[:, FH:]
    zpad = jnp.zeros((a.shape[0], 8), _f32)
    at0_ref[...] = jnp.concatenate(
        [a[:, 0:HH], a[:, 2 * HH:3 * HH], zpad], axis=1)
    at1_ref[...] = jnp.concatenate(
        [a[:, HH:2 * HH], a[:, 3 * HH:4 * HH], zpad], axis=1)


def _tc_embed_body(x_ref, w_ref, am_ref, xw0_ref, xw1_ref, at0_ref, at1_ref):
    xw = jnp.dot(x_ref[...], w_ref[...], preferred_element_type=_f32)
    a = jnp.dot(xw, am_ref[...], preferred_element_type=_f32)
    _split_outs(xw, a, xw0_ref, xw1_ref, at0_ref, at1_ref)


def _normalize(o_ref, d_ref, er_ref):
    den = jnp.dot(d_ref[...], er_ref[...], preferred_element_type=_f32)
    return o_ref[...] / (den + 1e-16)


def _tc_mid_body(o0_ref, o1_ref, d0_ref, d1_ref, er_ref, b_ref, w_ref, am_ref,
                 xw0_ref, xw1_ref, at0_ref, at1_ref):
    h0 = _normalize(o0_ref, d0_ref, er_ref) + b_ref[0:1, :FH]
    h1 = _normalize(o1_ref, d1_ref, er_ref) + b_ref[0:1, FH:]
    h0 = jnp.where(h0 > 0, h0, jnp.exp(h0) - 1.0)
    h1 = jnp.where(h1 > 0, h1, jnp.exp(h1) - 1.0)
    xw = (jnp.dot(h0, w_ref[:FH, :], preferred_element_type=_f32)
          + jnp.dot(h1, w_ref[FH:, :], preferred_element_type=_f32))
    a = jnp.dot(xw, am_ref[...], preferred_element_type=_f32)
    _split_outs(xw, a, xw0_ref, xw1_ref, at0_ref, at1_ref)


def _tc_out_body(o0_ref, o1_ref, d0_ref, d1_ref, er_ref, b_ref, wc_ref,
                 bc_ref, out_ref):
    h0 = _normalize(o0_ref, d0_ref, er_ref) + b_ref[0:1, :FH]
    h1 = _normalize(o1_ref, d1_ref, er_ref) + b_ref[0:1, FH:]
    out_ref[...] = (jnp.dot(h0, wc_ref[:FH, :], preferred_element_type=_f32)
                    + jnp.dot(h1, wc_ref[FH:, :], preferred_element_type=_f32)
                    + bc_ref[0:1, :])


def _row_spec(width):
    return pl.BlockSpec((BR, width), lambda i: (i, 0))


def _full_spec(shape):
    return pl.BlockSpec(shape, lambda i: (0, 0))


_CALLS = None


def _tc_calls():
    global _CALLS
    if _CALLS is not None:
        return _CALLS
    node_outs = ([jax.ShapeDtypeStruct((N, FH), _f32)] * 2
                 + [jax.ShapeDtypeStruct((N, DW), _f32)] * 2)
    node_out_specs = [_row_spec(FH)] * 2 + [_row_spec(DW)] * 2
    embed = pl.pallas_call(
        _tc_embed_body,
        grid=(NB,),
        in_specs=[_row_spec(F), _full_spec((F, F)), _full_spec((F, 2 * HEADS))],
        out_specs=node_out_specs,
        out_shape=node_outs,
    )
    mid = pl.pallas_call(
        _tc_mid_body,
        grid=(NB,),
        in_specs=[_row_spec(FH), _row_spec(FH), _row_spec(DW), _row_spec(DW),
                  _full_spec((DW, FH)), _full_spec((1, F)),
                  _full_spec((F, F)), _full_spec((F, 2 * HEADS))],
        out_specs=node_out_specs,
        out_shape=node_outs,
    )
    outk = pl.pallas_call(
        _tc_out_body,
        grid=(NB,),
        in_specs=[_row_spec(FH), _row_spec(FH), _row_spec(DW), _row_spec(DW),
                  _full_spec((DW, FH)), _full_spec((1, F)),
                  _full_spec((F, NCLS)), _full_spec((1, NCLS))],
        out_specs=[_row_spec(NCLS)],
        out_shape=[jax.ShapeDtypeStruct((N, NCLS), _f32)],
    )
    _CALLS = (embed, mid, outk)
    return _CALLS


def _sc_edge_body(srcv2, dstv2, xw0, xw1, at0, at1, z64, z16,
                  out0, out1, den0, den1,
                  tb, ob, db, asp, sslab, dslab,
                  rows_a, dst_a, asr_a, adr_a,
                  rows_b, dst_b, asr_b, adr_b,
                  gsem_a, gsem_b, ssem_a, ssem_b):
    c = lax.axis_index("c")
    s = lax.axis_index("s")
    r0 = s * RPT

    rowss = (rows_a, rows_b)
    dsts = (dst_a, dst_b)
    asrs = (asr_a, asr_b)
    adrs = (adr_a, adr_b)
    gsems = (gsem_a, gsem_b)
    ssems = (ssem_a, ssem_b)

    def part_copy(src, dst):
        pltpu.sync_copy(src.at[pl.ds(r0, RPT)], dst.at[pl.ds(r0, RPT)])

        @pl.when(s == NS - 1)
        def _():
            pltpu.sync_copy(src.at[pl.ds(NS * RPT, NTAIL)],
                            dst.at[pl.ds(NS * RPT, NTAIL)])

    @pl.when(c == 0)
    def _():
        part_copy(xw0, tb)
        part_copy(at0, asp)

    @pl.when(c == 1)
    def _():
        part_copy(xw1, tb)
        part_copy(at1, asp)

    part_copy(z64, ob)
    part_copy(z16, db)

    plsc.subcore_barrier()

    lanes = lax.iota(jnp.int32, 16)
    shift4 = (lanes + HH) & 15
    lmask = lanes < HH

    blk0 = s * MAIN

    def start_gathers(sl, b):
        pltpu.async_copy(tb.at[sslab.at[b]], rowss[sl], gsems[sl])
        pltpu.async_copy(asp.at[sslab.at[b]], asrs[sl], gsems[sl])
        pltpu.async_copy(asp.at[dslab.at[b]], adrs[sl], gsems[sl])

    def wait_gathers(sl, b):
        pltpu.make_async_copy(tb.at[sslab.at[b]], rowss[sl],
                              gsems[sl]).wait()
        pltpu.make_async_copy(asp.at[sslab.at[b]], asrs[sl],
                              gsems[sl]).wait()
        pltpu.make_async_copy(asp.at[dslab.at[b]], adrs[sl],
                              gsems[sl]).wait()

    def start_scatter(sl, b):
        pltpu.async_copy(rowss[sl], ob.at[dslab.at[b]], ssems[sl], add=True)
        pltpu.async_copy(dsts[sl], db.at[dslab.at[b]], ssems[sl], add=True)

    def wait_scatter(sl, b):
        pltpu.make_async_copy(rowss[sl], ob.at[dslab.at[b]],
                              ssems[sl]).wait()
        pltpu.make_async_copy(dsts[sl], db.at[dslab.at[b]],
                              ssems[sl]).wait()

    def compute(sl):
        rows_s = rowss[sl]
        dst_s = dsts[sl]
        asr_s = asrs[sl]
        adr_s = adrs[sl]

        def scale_body(k, acc):
            al = asr_s[k, :] + _vtake(adr_s[k, :], shift4)
            al = jnp.where(al > 0, al, al * 0.2)
            ev = jnp.exp(al)
            dst_s[k, :] = jnp.where(lmask, ev, 0.0)
            for h in range(HH):
                bh = _vtake(ev, jnp.full((16,), h, jnp.int32))
                rows_s[k, pl.ds(h * HID, HID)] = (
                    rows_s[k, pl.ds(h * HID, HID)] * bh)
            return acc

        lax.fori_loop(0, K, scale_body, 0)

    def round_body(r, carry):
        base = blk0 + SLAB * r
        pltpu.sync_copy(srcv2.at[pl.ds(base, SLAB)], sslab)
        pltpu.sync_copy(dstv2.at[pl.ds(base, SLAB)], dslab)
        start_gathers(0, 0)
        start_gathers(1, 1)
        for b in range(SLAB):
            sl = b % NSLOT
            wait_gathers(sl, b)
            compute(sl)
            start_scatter(sl, b)
            if b + NSLOT < SLAB:
                wait_scatter(sl, b)
                start_gathers(sl, b + NSLOT)
        wait_scatter(0, SLAB - 2)
        wait_scatter(1, SLAB - 1)
        return carry

    lax.fori_loop(0, MAIN // SLAB, round_body, 0)

    @pl.when(s < EXTRA)
    def _():
        blk = NS * MAIN + s
        pltpu.sync_copy(srcv2.at[pl.ds(blk, 1)], sslab.at[pl.ds(0, 1)])
        pltpu.sync_copy(dstv2.at[pl.ds(blk, 1)], dslab.at[pl.ds(0, 1)])
        pltpu.sync_copy(tb.at[sslab.at[0]], rows_a)
        pltpu.sync_copy(asp.at[sslab.at[0]], asr_a)
        pltpu.sync_copy(asp.at[dslab.at[0]], adr_a)
        compute(0)
        pltpu.sync_copy(rows_a, ob.at[dslab.at[0]], add=True)
        pltpu.sync_copy(dst_a, db.at[dslab.at[0]], add=True)

    plsc.subcore_barrier()

    @pl.when(c == 0)
    def _():
        part_copy(ob, out0)
        part_copy(db, den0)

    @pl.when(c == 1)
    def _():
        part_copy(ob, out1)
        part_copy(db, den1)


_SC_CALL = None


def _sc_call():
    global _SC_CALL
    if _SC_CALL is None:
        mesh = plsc.VectorSubcoreMesh(core_axis_name="c", subcore_axis_name="s")
        slot_bufs = [
            pltpu.VMEM((K, FH), _f32),
            pltpu.VMEM((K, DW), _f32),
            pltpu.VMEM((K, DW), _f32),
            pltpu.VMEM((K, DW), _f32),
        ]
        _SC_CALL = pl.kernel(
            _sc_edge_body,
            out_type=[jax.ShapeDtypeStruct((N, FH), _f32),
                      jax.ShapeDtypeStruct((N, FH), _f32),
                      jax.ShapeDtypeStruct((N, DW), _f32),
                      jax.ShapeDtypeStruct((N, DW), _f32)],
            mesh=mesh,
            compiler_params=pltpu.CompilerParams(use_tc_tiling_on_sc=False),
            scratch_types=(
                [pltpu.VMEM_SHARED((N, FH), _f32),
                 pltpu.VMEM_SHARED((N, FH), _f32),
                 pltpu.VMEM_SHARED((N, DW), _f32),
                 pltpu.VMEM_SHARED((N, DW), _f32),
                 pltpu.VMEM((SLAB, K), jnp.int32),
                 pltpu.VMEM((SLAB, K), jnp.int32)]
                + slot_bufs + slot_bufs
                + [pltpu.SemaphoreType.DMA, pltpu.SemaphoreType.DMA,
                   pltpu.SemaphoreType.DMA, pltpu.SemaphoreType.DMA]
            ),
        )
    return _SC_CALL


def _att_mats(att_src, att_dst):
    eye = jnp.eye(HEADS, dtype=_f32)
    a_s = (att_src[0][:, :, None] * eye[:, None, :]).reshape(F, HEADS)
    a_d = (att_dst[0][:, :, None] * eye[:, None, :]).reshape(F, HEADS)
    return jnp.concatenate([a_s, a_d], axis=1)


@jax.jit
def kernel(x, edge_index, W0, att_src0, att_dst0, b0,
           W1, att_src1, att_dst1, b1, Wc, bc):
    src = edge_index[0]
    dst = edge_index[1]
    am0 = _att_mats(att_src0, att_dst0)
    am1 = _att_mats(att_src1, att_dst1)
    er = jnp.concatenate(
        [jnp.kron(jnp.eye(HH, dtype=_f32), jnp.ones((1, HID), _f32)),
         jnp.zeros((DW - HH, FH), _f32)], axis=0)
    z64 = jnp.zeros((N, FH), _f32)
    z16 = jnp.zeros((N, DW), _f32)

    embed, mid, outk = _tc_calls()
    sc_edge = _sc_call()

    src2 = src.reshape(NBLK, K)
    dst2 = dst.reshape(NBLK, K)

    xw0a, xw0b, at0a, at0b = embed(x, W0, am0)
    o0a, o0b, d0a, d0b = sc_edge(src2, dst2, xw0a, xw0b, at0a, at0b,
                                 z64, z16)
    xw1a, xw1b, at1a, at1b = mid(o0a, o0b, d0a, d0b, er, b0[None, :],
                                 W1, am1)
    o1a, o1b, d1a, d1b = sc_edge(src2, dst2, xw1a, xw1b, at1a, at1b,
                                 z64, z16)
    (logits,) = outk(o1a, o1b, d1a, d1b, er, b1[None, :], Wc, bc[None, :])
    return logits


# speedup vs baseline: 105.1256x; 1.7280x over previous
import jax
import jax.numpy as jnp
from jax import lax
from jax.experimental import pallas as pl
from jax.experimental.pallas import tpu as pltpu
from jax.experimental.pallas import tpu_sc as plsc

N = 10000
E = 320000
F = 128
HEADS = 8
HID = 16
NCLS = 40
FH = 64
HH = 4
DW = 16
NB = 10
BR = N // NB
K = 128
NBLK = E // K
NS = 16
MAIN = NBLK // NS
EXTRA = NBLK - MAIN * NS
NSLOT = 2
SLAB = 6
RPT = 624
NTAIL = N - NS * RPT

_f32 = jnp.float32


def _vtake(v, idx):
    dn = lax.GatherDimensionNumbers(offset_dims=(), collapsed_slice_dims=(0,),
                                    start_index_map=(0,))
    return lax.gather(v, idx[:, None], dn, slice_sizes=(1,),
                      mode=lax.GatherScatterMode.PROMISE_IN_BOUNDS)


def _split_outs(xw, a, xw0_ref, xw1_ref, at0_ref, at1_ref):
    xw0_ref[...] = xw[:, :FH]
    xw1_ref[...] = xw[:, FH:]
    zpad = jnp.zeros((a.shape[0], 8), _f32)
    at0_ref[...] = jnp.concatenate(
        [a[:, 0:HH], a[:, 2 * HH:3 * HH], zpad], axis=1)
    at1_ref[...] = jnp.concatenate(
        [a[:, HH:2 * HH], a[:, 3 * HH:4 * HH], zpad], axis=1)


def _tc_embed_body(x_ref, w_ref, am_ref, xw0_ref, xw1_ref, at0_ref, at1_ref):
    xw = jnp.dot(x_ref[...], w_ref[...], preferred_element_type=_f32)
    a = jnp.dot(xw, am_ref[...], preferred_element_type=_f32)
    _split_outs(xw, a, xw0_ref, xw1_ref, at0_ref, at1_ref)


def _normalize(o_ref, d_ref, er_ref):
    den = jnp.dot(d_ref[...], er_ref[...], preferred_element_type=_f32)
    return o_ref[...] / (den + 1e-16)


def _tc_mid_body(o0_ref, o1_ref, d0_ref, d1_ref, er_ref, b_ref, w_ref, am_ref,
                 xw0_ref, xw1_ref, at0_ref, at1_ref):
    h0 = _normalize(o0_ref, d0_ref, er_ref) + b_ref[0:1, :FH]
    h1 = _normalize(o1_ref, d1_ref, er_ref) + b_ref[0:1, FH:]
    h0 = jnp.where(h0 > 0, h0, jnp.exp(h0) - 1.0)
    h1 = jnp.where(h1 > 0, h1, jnp.exp(h1) - 1.0)
    xw = (jnp.dot(h0, w_ref[:FH, :], preferred_element_type=_f32)
          + jnp.dot(h1, w_ref[FH:, :], preferred_element_type=_f32))
    a = jnp.dot(xw, am_ref[...], preferred_element_type=_f32)
    _split_outs(xw, a, xw0_ref, xw1_ref, at0_ref, at1_ref)


def _tc_out_body(o0_ref, o1_ref, d0_ref, d1_ref, er_ref, b_ref, wc_ref,
                 bc_ref, out_ref):
    h0 = _normalize(o0_ref, d0_ref, er_ref) + b_ref[0:1, :FH]
    h1 = _normalize(o1_ref, d1_ref, er_ref) + b_ref[0:1, FH:]
    out_ref[...] = (jnp.dot(h0, wc_ref[:FH, :], preferred_element_type=_f32)
                    + jnp.dot(h1, wc_ref[FH:, :], preferred_element_type=_f32)
                    + bc_ref[0:1, :])


def _row_spec(width):
    return pl.BlockSpec((BR, width), lambda i: (i, 0))


def _full_spec(shape):
    return pl.BlockSpec(shape, lambda i: (0, 0))


_CALLS = None


def _tc_calls():
    global _CALLS
    if _CALLS is not None:
        return _CALLS
    node_outs = ([jax.ShapeDtypeStruct((N, FH), _f32)] * 2
                 + [jax.ShapeDtypeStruct((N, DW), _f32)] * 2)
    node_out_specs = [_row_spec(FH)] * 2 + [_row_spec(DW)] * 2
    embed = pl.pallas_call(
        _tc_embed_body,
        grid=(NB,),
        in_specs=[_row_spec(F), _full_spec((F, F)), _full_spec((F, 2 * HEADS))],
        out_specs=node_out_specs,
        out_shape=node_outs,
    )
    mid = pl.pallas_call(
        _tc_mid_body,
        grid=(NB,),
        in_specs=[_row_spec(FH), _row_spec(FH), _row_spec(DW), _row_spec(DW),
                  _full_spec((DW, FH)), _full_spec((1, F)),
                  _full_spec((F, F)), _full_spec((F, 2 * HEADS))],
        out_specs=node_out_specs,
        out_shape=node_outs,
    )
    outk = pl.pallas_call(
        _tc_out_body,
        grid=(NB,),
        in_specs=[_row_spec(FH), _row_spec(FH), _row_spec(DW), _row_spec(DW),
                  _full_spec((DW, FH)), _full_spec((1, F)),
                  _full_spec((F, NCLS)), _full_spec((1, NCLS))],
        out_specs=[_row_spec(NCLS)],
        out_shape=[jax.ShapeDtypeStruct((N, NCLS), _f32)],
    )
    _CALLS = (embed, mid, outk)
    return _CALLS


def _sc_edge_body(srcv2, dstv2, xw0, xw1, at0, at1, z64, z16,
                  out0, out1, den0, den1,
                  tb, ob, db, asp, sslab, dslab,
                  rows_a, dst_a, asr_a, adr_a,
                  rows_b, dst_b, asr_b, adr_b,
                  gsem_a, gsem_b, ssem_a, ssem_b):
    c = lax.axis_index("c")
    s = lax.axis_index("s")
    r0 = s * RPT

    rowss = (rows_a, rows_b)
    dsts = (dst_a, dst_b)
    asrs = (asr_a, asr_b)
    adrs = (adr_a, adr_b)
    gsems = (gsem_a, gsem_b)
    ssems = (ssem_a, ssem_b)

    def part_copy(src, dst):
        pltpu.sync_copy(src.at[pl.ds(r0, RPT)], dst.at[pl.ds(r0, RPT)])

        @pl.when(s == NS - 1)
        def _():
            pltpu.sync_copy(src.at[pl.ds(NS * RPT, NTAIL)],
                            dst.at[pl.ds(NS * RPT, NTAIL)])

    @pl.when(c == 0)
    def _():
        part_copy(xw0, tb)
        part_copy(at0, asp)

    @pl.when(c == 1)
    def _():
        part_copy(xw1, tb)
        part_copy(at1, asp)

    part_copy(z64, ob)
    part_copy(z16, db)

    plsc.subcore_barrier()

    lanes = lax.iota(jnp.int32, 16)
    shift4 = (lanes + HH) & 15
    lmask = lanes < HH

    blk0 = s * MAIN

    def start_gathers(sl, b):
        pltpu.async_copy(tb.at[sslab.at[b]], rowss[sl], gsems[sl])
        pltpu.async_copy(asp.at[sslab.at[b]], asrs[sl], gsems[sl])
        pltpu.async_copy(asp.at[dslab.at[b]], adrs[sl], gsems[sl])

    def wait_gathers(sl, b):
        pltpu.make_async_copy(tb.at[sslab.at[b]], rowss[sl],
                              gsems[sl]).wait()
        pltpu.make_async_copy(asp.at[sslab.at[b]], asrs[sl],
                              gsems[sl]).wait()
        pltpu.make_async_copy(asp.at[dslab.at[b]], adrs[sl],
                              gsems[sl]).wait()

    def start_scatter(sl, b):
        pltpu.async_copy(rowss[sl], ob.at[dslab.at[b]], ssems[sl], add=True)
        pltpu.async_copy(dsts[sl], db.at[dslab.at[b]], ssems[sl], add=True)

    def wait_scatter(sl, b):
        pltpu.make_async_copy(rowss[sl], ob.at[dslab.at[b]],
                              ssems[sl]).wait()
        pltpu.make_async_copy(dsts[sl], db.at[dslab.at[b]],
                              ssems[sl]).wait()

    def compute(sl):
        rows_s = rowss[sl]
        dst_s = dsts[sl]
        asr_s = asrs[sl]
        adr_s = adrs[sl]

        @plsc.parallel_loop(0, K, unroll=4)
        def scale_body(k):
            al = asr_s[k, :] + _vtake(adr_s[k, :], shift4)
            al = jnp.where(al > 0, al, al * 0.2)
            ev = jnp.exp(al)
            dst_s[k, :] = jnp.where(lmask, ev, 0.0)
            for h in range(HH):
                bh = _vtake(ev, jnp.full((16,), h, jnp.int32))
                rows_s[k, pl.ds(h * HID, HID)] = (
                    rows_s[k, pl.ds(h * HID, HID)] * bh)

    def round_body(r, carry):
        base = blk0 + SLAB * r
        pltpu.sync_copy(srcv2.at[pl.ds(base, SLAB)], sslab)
        pltpu.sync_copy(dstv2.at[pl.ds(base, SLAB)], dslab)
        start_gathers(0, 0)
        start_gathers(1, 1)
        for b in range(SLAB):
            sl = b % NSLOT
            wait_gathers(sl, b)
            compute(sl)
            start_scatter(sl, b)
            if b + NSLOT < SLAB:
                wait_scatter(sl, b)
                start_gathers(sl, b + NSLOT)
        wait_scatter(0, SLAB - 2)
        wait_scatter(1, SLAB - 1)
        return carry

    lax.fori_loop(0, MAIN // SLAB, round_body, 0)

    @pl.when(s < EXTRA)
    def _():
        blk = NS * MAIN + s
        pltpu.sync_copy(srcv2.at[pl.ds(blk, 1)], sslab.at[pl.ds(0, 1)])
        pltpu.sync_copy(dstv2.at[pl.ds(blk, 1)], dslab.at[pl.ds(0, 1)])
        pltpu.sync_copy(tb.at[sslab.at[0]], rows_a)
        pltpu.sync_copy(asp.at[sslab.at[0]], asr_a)
        pltpu.sync_copy(asp.at[dslab.at[0]], adr_a)
        compute(0)
        pltpu.sync_copy(rows_a, ob.at[dslab.at[0]], add=True)
        pltpu.sync_copy(dst_a, db.at[dslab.at[0]], add=True)

    plsc.subcore_barrier()

    @pl.when(c == 0)
    def _():
        part_copy(ob, out0)
        part_copy(db, den0)

    @pl.when(c == 1)
    def _():
        part_copy(ob, out1)
        part_copy(db, den1)


_SC_CALL = None


def _sc_call():
    global _SC_CALL
    if _SC_CALL is None:
        mesh = plsc.VectorSubcoreMesh(core_axis_name="c", subcore_axis_name="s")
        slot_bufs = [
            pltpu.VMEM((K, FH), _f32),
            pltpu.VMEM((K, DW), _f32),
            pltpu.VMEM((K, DW), _f32),
            pltpu.VMEM((K, DW), _f32),
        ]
        _SC_CALL = pl.kernel(
            _sc_edge_body,
            out_type=[jax.ShapeDtypeStruct((N, FH), _f32),
                      jax.ShapeDtypeStruct((N, FH), _f32),
                      jax.ShapeDtypeStruct((N, DW), _f32),
                      jax.ShapeDtypeStruct((N, DW), _f32)],
            mesh=mesh,
            compiler_params=pltpu.CompilerParams(use_tc_tiling_on_sc=False),
            scratch_types=(
                [pltpu.VMEM_SHARED((N, FH), _f32),
                 pltpu.VMEM_SHARED((N, FH), _f32),
                 pltpu.VMEM_SHARED((N, DW), _f32),
                 pltpu.VMEM_SHARED((N, DW), _f32),
                 pltpu.VMEM((SLAB, K), jnp.int32),
                 pltpu.VMEM((SLAB, K), jnp.int32)]
                + slot_bufs + slot_bufs
                + [pltpu.SemaphoreType.DMA, pltpu.SemaphoreType.DMA,
                   pltpu.SemaphoreType.DMA, pltpu.SemaphoreType.DMA]
            ),
        )
    return _SC_CALL


def _att_mats(att_src, att_dst):
    eye = jnp.eye(HEADS, dtype=_f32)
    a_s = (att_src[0][:, :, None] * eye[:, None, :]).reshape(F, HEADS)
    a_d = (att_dst[0][:, :, None] * eye[:, None, :]).reshape(F, HEADS)
    return jnp.concatenate([a_s, a_d], axis=1)


@jax.jit
def kernel(x, edge_index, W0, att_src0, att_dst0, b0,
           W1, att_src1, att_dst1, b1, Wc, bc):
    src = edge_index[0]
    dst = edge_index[1]
    am0 = _att_mats(att_src0, att_dst0)
    am1 = _att_mats(att_src1, att_dst1)
    er = jnp.concatenate(
        [jnp.kron(jnp.eye(HH, dtype=_f32), jnp.ones((1, HID), _f32)),
         jnp.zeros((DW - HH, FH), _f32)], axis=0)
    z64 = jnp.zeros((N, FH), _f32)
    z16 = jnp.zeros((N, DW), _f32)

    embed, mid, outk = _tc_calls()
    sc_edge = _sc_call()

    src2 = src.reshape(NBLK, K)
    dst2 = dst.reshape(NBLK, K)

    xw0a, xw0b, at0a, at0b = embed(x, W0, am0)
    o0a, o0b, d0a, d0b = sc_edge(src2, dst2, xw0a, xw0b, at0a, at0b,
                                 z64, z16)
    xw1a, xw1b, at1a, at1b = mid(o0a, o0b, d0a, d0b, er, b0[None, :],
                                 W1, am1)
    o1a, o1b, d1a, d1b = sc_edge(src2, dst2, xw1a, xw1b, at1a, at1b,
                                 z64, z16)
    (logits,) = outk(o1a, o1b, d1a, d1b, er, b1[None, :], Wc, bc[None, :])
    return logits


# speedup vs baseline: 106.8776x; 1.0167x over previous
import jax
import jax.numpy as jnp
from jax import lax
from jax.experimental import pallas as pl
from jax.experimental.pallas import tpu as pltpu
from jax.experimental.pallas import tpu_sc as plsc

N = 10000
E = 320000
F = 128
HEADS = 8
HID = 16
NCLS = 40
FH = 64
HH = 4
DW = 16
NB = 10
BR = N // NB
K = 128
NBLK = E // K
NS = 16
MAIN = NBLK // NS
EXTRA = NBLK - MAIN * NS
NSLOT = 2
SLAB = 6
RPT = 624
NTAIL = N - NS * RPT

_f32 = jnp.float32


def _vtake(v, idx):
    dn = lax.GatherDimensionNumbers(offset_dims=(), collapsed_slice_dims=(0,),
                                    start_index_map=(0,))
    return lax.gather(v, idx[:, None], dn, slice_sizes=(1,),
                      mode=lax.GatherScatterMode.PROMISE_IN_BOUNDS)


def _split_outs(xw, a, xw0_ref, xw1_ref, at0_ref, at1_ref):
    xw0_ref[...] = xw[:, :FH]
    xw1_ref[...] = xw[:, FH:]
    zpad = jnp.zeros((a.shape[0], 8), _f32)
    at0_ref[...] = jnp.concatenate(
        [a[:, 0:HH], a[:, 2 * HH:3 * HH], zpad], axis=1)
    at1_ref[...] = jnp.concatenate(
        [a[:, HH:2 * HH], a[:, 3 * HH:4 * HH], zpad], axis=1)


def _tc_embed_body(x_ref, w_ref, am_ref, xw0_ref, xw1_ref, at0_ref, at1_ref):
    xw = jnp.dot(x_ref[...], w_ref[...], preferred_element_type=_f32)
    a = jnp.dot(xw, am_ref[...], preferred_element_type=_f32)
    _split_outs(xw, a, xw0_ref, xw1_ref, at0_ref, at1_ref)


def _normalize(o_ref, d_ref, er_ref):
    den = jnp.dot(d_ref[...], er_ref[...], preferred_element_type=_f32)
    return o_ref[...] / (den + 1e-16)


def _tc_mid_body(o0_ref, o1_ref, d0_ref, d1_ref, er_ref, b_ref, w_ref, am_ref,
                 xw0_ref, xw1_ref, at0_ref, at1_ref):
    h0 = _normalize(o0_ref, d0_ref, er_ref) + b_ref[0:1, :FH]
    h1 = _normalize(o1_ref, d1_ref, er_ref) + b_ref[0:1, FH:]
    h0 = jnp.where(h0 > 0, h0, jnp.exp(h0) - 1.0)
    h1 = jnp.where(h1 > 0, h1, jnp.exp(h1) - 1.0)
    xw = (jnp.dot(h0, w_ref[:FH, :], preferred_element_type=_f32)
          + jnp.dot(h1, w_ref[FH:, :], preferred_element_type=_f32))
    a = jnp.dot(xw, am_ref[...], preferred_element_type=_f32)
    _split_outs(xw, a, xw0_ref, xw1_ref, at0_ref, at1_ref)


def _tc_out_body(o0_ref, o1_ref, d0_ref, d1_ref, er_ref, b_ref, wc_ref,
                 bc_ref, out_ref):
    h0 = _normalize(o0_ref, d0_ref, er_ref) + b_ref[0:1, :FH]
    h1 = _normalize(o1_ref, d1_ref, er_ref) + b_ref[0:1, FH:]
    out_ref[...] = (jnp.dot(h0, wc_ref[:FH, :], preferred_element_type=_f32)
                    + jnp.dot(h1, wc_ref[FH:, :], preferred_element_type=_f32)
                    + bc_ref[0:1, :])


def _row_spec(width):
    return pl.BlockSpec((BR, width), lambda i: (i, 0))


def _full_spec(shape):
    return pl.BlockSpec(shape, lambda i: (0, 0))


_CALLS = None


def _tc_calls():
    global _CALLS
    if _CALLS is not None:
        return _CALLS
    node_outs = ([jax.ShapeDtypeStruct((N, FH), _f32)] * 2
                 + [jax.ShapeDtypeStruct((N, DW), _f32)] * 2)
    node_out_specs = [_row_spec(FH)] * 2 + [_row_spec(DW)] * 2
    embed = pl.pallas_call(
        _tc_embed_body,
        grid=(NB,),
        in_specs=[_row_spec(F), _full_spec((F, F)), _full_spec((F, 2 * HEADS))],
        out_specs=node_out_specs,
        out_shape=node_outs,
    )
    mid = pl.pallas_call(
        _tc_mid_body,
        grid=(NB,),
        in_specs=[_row_spec(FH), _row_spec(FH), _row_spec(DW), _row_spec(DW),
                  _full_spec((DW, FH)), _full_spec((1, F)),
                  _full_spec((F, F)), _full_spec((F, 2 * HEADS))],
        out_specs=node_out_specs,
        out_shape=node_outs,
    )
    outk = pl.pallas_call(
        _tc_out_body,
        grid=(NB,),
        in_specs=[_row_spec(FH), _row_spec(FH), _row_spec(DW), _row_spec(DW),
                  _full_spec((DW, FH)), _full_spec((1, F)),
                  _full_spec((F, NCLS)), _full_spec((1, NCLS))],
        out_specs=[_row_spec(NCLS)],
        out_shape=[jax.ShapeDtypeStruct((N, NCLS), _f32)],
    )
    _CALLS = (embed, mid, outk)
    return _CALLS


def _sc_edge_body(srcv2, dstv2, xw0, xw1, at0, at1, z64, z16,
                  out0, out1, den0, den1,
                  tb, ob, db, asp, sslab, dslab,
                  rows_a, dst_a, asr_a, adr_a,
                  rows_b, dst_b, asr_b, adr_b,
                  gsem_a, gsem_b, ssem_a, ssem_b):
    c = lax.axis_index("c")
    s = lax.axis_index("s")
    r0 = s * RPT

    rowss = (rows_a, rows_b)
    dsts = (dst_a, dst_b)
    asrs = (asr_a, asr_b)
    adrs = (adr_a, adr_b)
    gsems = (gsem_a, gsem_b)
    ssems = (ssem_a, ssem_b)

    def part_copy(src, dst):
        pltpu.sync_copy(src.at[pl.ds(r0, RPT)], dst.at[pl.ds(r0, RPT)])

        @pl.when(s == NS - 1)
        def _():
            pltpu.sync_copy(src.at[pl.ds(NS * RPT, NTAIL)],
                            dst.at[pl.ds(NS * RPT, NTAIL)])

    @pl.when(c == 0)
    def _():
        part_copy(xw0, tb)
        part_copy(at0, asp)

    @pl.when(c == 1)
    def _():
        part_copy(xw1, tb)
        part_copy(at1, asp)

    part_copy(z64, ob)
    part_copy(z16, db)

    plsc.subcore_barrier()

    lanes = lax.iota(jnp.int32, 16)
    shift4 = (lanes + HH) & 15
    lmask = lanes < HH

    blk0 = s * MAIN

    def start_gathers(sl, b):
        pltpu.async_copy(tb.at[sslab.at[b]], rowss[sl], gsems[sl])
        pltpu.async_copy(asp.at[sslab.at[b]], asrs[sl], gsems[sl])
        pltpu.async_copy(asp.at[dslab.at[b]], adrs[sl], gsems[sl])

    def wait_gathers(sl, b):
        pltpu.make_async_copy(tb.at[sslab.at[b]], rowss[sl],
                              gsems[sl]).wait()
        pltpu.make_async_copy(asp.at[sslab.at[b]], asrs[sl],
                              gsems[sl]).wait()
        pltpu.make_async_copy(asp.at[dslab.at[b]], adrs[sl],
                              gsems[sl]).wait()

    def start_scatter(sl, b):
        pltpu.async_copy(rowss[sl], ob.at[dslab.at[b]], ssems[sl], add=True)
        pltpu.async_copy(dsts[sl], db.at[dslab.at[b]], ssems[sl], add=True)

    def wait_scatter(sl, b):
        pltpu.make_async_copy(rowss[sl], ob.at[dslab.at[b]],
                              ssems[sl]).wait()
        pltpu.make_async_copy(dsts[sl], db.at[dslab.at[b]],
                              ssems[sl]).wait()

    def compute(sl):
        rows_s = rowss[sl]
        dst_s = dsts[sl]
        asr_s = asrs[sl]
        adr_s = adrs[sl]

        @plsc.parallel_loop(0, K, unroll=8)
        def scale_body(k):
            al = asr_s[k, :] + _vtake(adr_s[k, :], shift4)
            al = jnp.where(al > 0, al, al * 0.2)
            ev = jnp.exp(al)
            dst_s[k, :] = jnp.where(lmask, ev, 0.0)
            for h in range(HH):
                bh = _vtake(ev, jnp.full((16,), h, jnp.int32))
                rows_s[k, pl.ds(h * HID, HID)] = (
                    rows_s[k, pl.ds(h * HID, HID)] * bh)

    def round_body(r, carry):
        base = blk0 + SLAB * r
        pltpu.sync_copy(srcv2.at[pl.ds(base, SLAB)], sslab)
        pltpu.sync_copy(dstv2.at[pl.ds(base, SLAB)], dslab)
        start_gathers(0, 0)
        start_gathers(1, 1)
        for b in range(SLAB):
            sl = b % NSLOT
            wait_gathers(sl, b)
            compute(sl)
            start_scatter(sl, b)
            if b + NSLOT < SLAB:
                wait_scatter(sl, b)
                start_gathers(sl, b + NSLOT)
        wait_scatter(0, SLAB - 2)
        wait_scatter(1, SLAB - 1)
        return carry

    lax.fori_loop(0, MAIN // SLAB, round_body, 0)

    @pl.when(s < EXTRA)
    def _():
        blk = NS * MAIN + s
        pltpu.sync_copy(srcv2.at[pl.ds(blk, 1)], sslab.at[pl.ds(0, 1)])
        pltpu.sync_copy(dstv2.at[pl.ds(blk, 1)], dslab.at[pl.ds(0, 1)])
        pltpu.sync_copy(tb.at[sslab.at[0]], rows_a)
        pltpu.sync_copy(asp.at[sslab.at[0]], asr_a)
        pltpu.sync_copy(asp.at[dslab.at[0]], adr_a)
        compute(0)
        pltpu.sync_copy(rows_a, ob.at[dslab.at[0]], add=True)
        pltpu.sync_copy(dst_a, db.at[dslab.at[0]], add=True)

    plsc.subcore_barrier()

    @pl.when(c == 0)
    def _():
        part_copy(ob, out0)
        part_copy(db, den0)

    @pl.when(c == 1)
    def _():
        part_copy(ob, out1)
        part_copy(db, den1)


_SC_CALL = None


def _sc_call():
    global _SC_CALL
    if _SC_CALL is None:
        mesh = plsc.VectorSubcoreMesh(core_axis_name="c", subcore_axis_name="s")
        slot_bufs = [
            pltpu.VMEM((K, FH), _f32),
            pltpu.VMEM((K, DW), _f32),
            pltpu.VMEM((K, DW), _f32),
            pltpu.VMEM((K, DW), _f32),
        ]
        _SC_CALL = pl.kernel(
            _sc_edge_body,
            out_type=[jax.ShapeDtypeStruct((N, FH), _f32),
                      jax.ShapeDtypeStruct((N, FH), _f32),
                      jax.ShapeDtypeStruct((N, DW), _f32),
                      jax.ShapeDtypeStruct((N, DW), _f32)],
            mesh=mesh,
            compiler_params=pltpu.CompilerParams(use_tc_tiling_on_sc=False),
            scratch_types=(
                [pltpu.VMEM_SHARED((N, FH), _f32),
                 pltpu.VMEM_SHARED((N, FH), _f32),
                 pltpu.VMEM_SHARED((N, DW), _f32),
                 pltpu.VMEM_SHARED((N, DW), _f32),
                 pltpu.VMEM((SLAB, K), jnp.int32),
                 pltpu.VMEM((SLAB, K), jnp.int32)]
                + slot_bufs + slot_bufs
                + [pltpu.SemaphoreType.DMA, pltpu.SemaphoreType.DMA,
                   pltpu.SemaphoreType.DMA, pltpu.SemaphoreType.DMA]
            ),
        )
    return _SC_CALL


def _att_mats(att_src, att_dst):
    eye = jnp.eye(HEADS, dtype=_f32)
    a_s = (att_src[0][:, :, None] * eye[:, None, :]).reshape(F, HEADS)
    a_d = (att_dst[0][:, :, None] * eye[:, None, :]).reshape(F, HEADS)
    return jnp.concatenate([a_s, a_d], axis=1)


@jax.jit
def kernel(x, edge_index, W0, att_src0, att_dst0, b0,
           W1, att_src1, att_dst1, b1, Wc, bc):
    src = edge_index[0]
    dst = edge_index[1]
    am0 = _att_mats(att_src0, att_dst0)
    am1 = _att_mats(att_src1, att_dst1)
    er = jnp.concatenate(
        [jnp.kron(jnp.eye(HH, dtype=_f32), jnp.ones((1, HID), _f32)),
         jnp.zeros((DW - HH, FH), _f32)], axis=0)
    z64 = jnp.zeros((N, FH), _f32)
    z16 = jnp.zeros((N, DW), _f32)

    embed, mid, outk = _tc_calls()
    sc_edge = _sc_call()

    src2 = src.reshape(NBLK, K)
    dst2 = dst.reshape(NBLK, K)

    xw0a, xw0b, at0a, at0b = embed(x, W0, am0)
    o0a, o0b, d0a, d0b = sc_edge(src2, dst2, xw0a, xw0b, at0a, at0b,
                                 z64, z16)
    xw1a, xw1b, at1a, at1b = mid(o0a, o0b, d0a, d0b, er, b0[None, :],
                                 W1, am1)
    o1a, o1b, d1a, d1b = sc_edge(src2, dst2, xw1a, xw1b, at1a, at1b,
                                 z64, z16)
    (logits,) = outk(o1a, o1b, d1a, d1b, er, b1[None, :], Wc, bc[None, :])
    return logits
